# baseline (speedup 1.0000x reference)
import functools

import jax
import jax.numpy as jnp
from jax.experimental import pallas as pl
from jax.experimental.pallas import tpu as pltpu

_IMG = 32
_IN_CH = 3
_NCLS = 4
_STAGE_CH = [[4, 4], [8, 8], [16, 16, 16, 16], [32, 32, 32, 32],
             [32, 32, 32, 32]]


def _r8(v):
    return ((v + 7) // 8) * 8


def _geometry():
    h = w = _IMG
    g0 = []
    ww = w
    for convs in _STAGE_CH:
        g0.append((len(convs) + 1) * (ww + 3))
        ww //= 2
    stages = []
    c = _IN_CH
    for si, convs in enumerate(_STAGE_CH):
        hp, wp = h + 2, w + 2
        g = g0[si]
        conv_geo = []
        for cout in convs:
            g -= wp + 1
            conv_geo.append(dict(cinp=_r8(c), coutp=_r8(cout),
                                 l_out=2 * g + hp * wp))
            c = cout
        last = si == len(_STAGE_CH) - 1
        h2, w2 = h // 2, w // 2
        cols = h2 * w2 if last else 2 * g0[si + 1] + (h2 + 2) * (w2 + 2)
        stages.append(dict(hp=hp, wp=wp, w=w, convs=conv_geo,
                           lm4=2 * g + hp * wp - (wp + 1),
                           cp=_r8(c), c=c, h2=h2, w2=w2, cols=cols,
                           g_out=None if last else g0[si + 1], last=last))
        h, w = h2, w2
    return g0[0], stages


def _fwd_kernel(*refs, stages, n_in, nb):
    ins, outs = refs[:n_in], refs[n_in:]
    p = ins[0][0]
    ri = 1
    for si, st in enumerate(stages):
        wp = st["wp"]
        for cg in st["convs"]:
            wref = ins[ri]
            b = ins[ri + 1][...]
            mask = ins[ri + 2][...]
            ri += 3
            lo = cg["l_out"]
            base = wp + 1
            acc = None
            for t in range(9):
                dy, dx = divmod(t, 3)
                s = base + (dy - 1) * wp + (dx - 1)
                prod = jnp.dot(wref[t], p[:, s:s + lo],
                               preferred_element_type=jnp.float32)
                acc = prod if acc is None else acc + prod
            acc = jnp.maximum(acc + b, 0.0)
            p = jnp.where(mask > 0.5, acc, 0.0)
        sref = ins[ri][...]
        bref = ins[ri + 1][...]
        ri += 2
        lm4 = st["lm4"]
        a0 = p[:, 0:lm4]
        a1 = p[:, 1:1 + lm4]
        a2 = p[:, wp:wp + lm4]
        a3 = p[:, wp + 1:wp + 1 + lm4]
        m01 = jnp.maximum(a0, a1)
        k01 = jnp.where(a1 > a0, 1.0, 0.0)
        m23 = jnp.maximum(a2, a3)
        k23 = jnp.where(a3 > a2, 3.0, 2.0)
        mk = jnp.concatenate(
            [jnp.maximum(m01, m23), jnp.where(m23 > m01, k23, k01)], axis=0)
        gat = jnp.dot(mk, sref, preferred_element_type=jnp.float32)
        cp = st["cp"] * nb
        vals, kk = gat[:cp], gat[cp:]
        dy = jnp.where(kk >= 1.5, 1.0, 0.0)
        dx = kk - 2.0 * dy
        idx = bref + dy * float(st["w"]) + dx
        outs[si][0] = (idx + 0.5).astype(jnp.int32)
        p = vals
    w1t, b1c, w2t, b2c, w3t, b3c = (ins[ri + j][...] for j in range(6))
    h = jnp.maximum(jnp.dot(w1t, p, preferred_element_type=jnp.float32)
                    + b1c, 0.0)
    h = jnp.maximum(jnp.dot(w2t, h, preferred_element_type=jnp.float32)
                    + b2c, 0.0)
    o = jnp.dot(w3t, h, preferred_element_type=jnp.float32) + b3c
    outs[5][0] = o.astype(outs[5].dtype)


def kernel(x, feat_0, feat_1, feat_2, feat_3, feat_4, feat_5, feat_6,
           feat_7, feat_8, feat_9, feat_10, feat_11, feat_12, feat_13,
           feat_14, feat_15, feat_16, feat_17, feat_18, feat_19, feat_20,
           feat_21, feat_22, feat_23, feat_24, feat_25, feat_26, feat_27,
           feat_28, feat_29, feat_30, feat_31, feat_32, feat_33, feat_34,
           feat_35, feat_36, feat_37, feat_38, feat_39, feat_40, feat_41,
           feat_42, feat_43, feat_44, feat_45, feat_46, feat_47, feat_48,
           feat_49, feat_50, feat_51, feat_52, feat_53, feat_54, feat_55,
           feat_56, feat_57, w1, b1, w2, b2, w3, b3):
    feats = [feat_0, feat_1, feat_2, feat_3, feat_4, feat_5, feat_6, feat_7,
             feat_8, feat_9, feat_10, feat_11, feat_12, feat_13, feat_14,
             feat_15, feat_16, feat_17, feat_18, feat_19, feat_20, feat_21,
             feat_22, feat_23, feat_24, feat_25, feat_26, feat_27, feat_28,
             feat_29, feat_30, feat_31, feat_32, feat_33, feat_34, feat_35,
             feat_36, feat_37, feat_38, feat_39, feat_40, feat_41, feat_42,
             feat_43, feat_44, feat_45, feat_46, feat_47, feat_48, feat_49,
             feat_50, feat_51, feat_52, feat_53, feat_54, feat_55, feat_56,
             feat_57]
    g0, stages = _geometry()
    n = x.shape[0]
    nb = 8
    eye = jnp.eye(nb, dtype=jnp.float32)

    def bdiag(m):
        r, c = m.shape
        return jnp.einsum("mn,rc->mrnc", eye, m).reshape(nb * r, nb * c)

    ops = []
    fi = 0
    for st in stages:
        for _ in st["convs"]:
            w9, b, mask = feats[fi], feats[fi + 1], feats[fi + 2]
            fi += 3
            coutp, cinp = w9.shape[1], w9.shape[2]
            wbd = jnp.einsum("mn,toc->tmonc", eye, w9).reshape(
                9, nb * coutp, nb * cinp)
            ops += [wbd, jnp.tile(b, (nb, 1)), mask]
        ops += [feats[fi], feats[fi + 1]]
        fi += 2
    ops += [bdiag(w1.T), jnp.tile(b1.reshape(-1, 1), (nb, 1)),
            bdiag(w2.T), jnp.tile(b2.reshape(-1, 1), (nb, 1)),
            bdiag(jnp.pad(w3.T, ((0, 8 - _NCLS), (0, 0)))),
            jnp.tile(jnp.pad(b3, (0, 8 - _NCLS)).reshape(-1, 1), (nb, 1))]
    n_in = 1 + len(ops)

    cp0 = _r8(_IN_CH)
    l0 = 2 * g0 + (_IMG + 2) * (_IMG + 2)
    xp = jnp.pad(x, ((0, 0), (0, cp0 - _IN_CH), (1, 1), (1, 1)))
    xp = jnp.pad(xp.reshape(n, cp0, (_IMG + 2) * (_IMG + 2)),
                 ((0, 0), (0, 0), (g0, g0)))
    xp = xp.reshape(n // nb, nb * cp0, l0)

    out_shapes, out_specs = [], []
    for st in stages:
        cpb = nb * st["cp"]
        out_shapes.append(
            jax.ShapeDtypeStruct((n // nb, cpb, st["cols"]), jnp.int32))
        out_specs.append(
            pl.BlockSpec((1, cpb, st["cols"]), lambda i: (i, 0, 0)))
    out_shapes.append(jax.ShapeDtypeStruct((n // nb, nb * 8, 1), jnp.float32))
    out_specs.append(pl.BlockSpec((1, nb * 8, 1), lambda i: (i, 0, 0)))

    in_specs = [pl.BlockSpec((1, nb * cp0, l0), lambda i: (i, 0, 0))]
    for a in ops:
        in_specs.append(
            pl.BlockSpec(a.shape, lambda i, nd=a.ndim: (0,) * nd))

    outs = pl.pallas_call(
        functools.partial(_fwd_kernel, stages=stages, n_in=n_in, nb=nb),
        grid=(n // nb,),
        out_shape=tuple(out_shapes),
        in_specs=in_specs,
        out_specs=tuple(out_specs),
        compiler_params=pltpu.CompilerParams(
            dimension_semantics=("parallel",)),
    )(xp, *ops)

    switch_indices = []
    for st, arr in zip(stages, outs[:5]):
        c, h2, w2 = st["c"], st["h2"], st["w2"]
        arr = arr.reshape(n, st["cp"], st["cols"])
        if st["last"]:
            switch_indices.append(arr[:, :c, :].reshape(n, c, h2, w2))
        else:
            go = st["g_out"]
            sp = (h2 + 2) * (w2 + 2)
            switch_indices.append(
                arr[:, :c, go:go + sp].reshape(n, c, h2 + 2, w2 + 2)
                [:, :, 1:1 + h2, 1:1 + w2])
    logits = outs[5].reshape(n, 8)[:, :_NCLS]
    return logits, switch_indices


# speedup vs baseline: 1.0194x; 1.0194x over previous
import functools

import jax
import jax.numpy as jnp
import numpy as np
from jax.experimental import pallas as pl
from jax.experimental.pallas import tpu as pltpu

_IMG = 32
_IN_CH = 3
_NCLS = 4
_STAGE_CH = [[4, 4], [8, 8], [16, 16, 16, 16], [32, 32, 32, 32],
             [32, 32, 32, 32]]


def _r8(v):
    return ((v + 7) // 8) * 8


def _geometry():
    h = w = _IMG
    g0 = []
    ww = w
    for convs in _STAGE_CH:
        g0.append((len(convs) + 1) * (ww + 3))
        ww //= 2
    stages = []
    c = _IN_CH
    for si, convs in enumerate(_STAGE_CH):
        hp, wp = h + 2, w + 2
        g = g0[si]
        conv_geo = []
        for cout in convs:
            g -= wp + 1
            conv_geo.append(dict(cinp=_r8(c), coutp=_r8(cout),
                                 l_out=2 * g + hp * wp))
            c = cout
        last = si == len(_STAGE_CH) - 1
        h2, w2 = h // 2, w // 2
        stages.append(dict(hp=hp, wp=wp, h=h, w=w, convs=conv_geo,
                           g_fin=g, lm4=2 * g + hp * wp - (wp + 1),
                           cp=_r8(c), c=c, h2=h2, w2=w2, last=last))
        h, w = h2, w2
    return g0[0], stages


def _dense_gather_np(h, w, g):
    hp, wp = h + 2, w + 2
    h2, w2 = h // 2, w // 2
    s = np.zeros((2 * g + hp * wp - (wp + 1), h2 * w2), np.float32)
    for oh in range(h2):
        for ow in range(w2):
            s[g + (2 * oh + 1) * wp + (2 * ow + 1), oh * w2 + ow] = 1.0
    return s


def _dense_base_np(h, w):
    h2, w2 = h // 2, w // 2
    return ((np.arange(h2) * 2 * w)[:, None]
            + np.arange(w2) * 2).astype(np.float32).reshape(1, h2 * w2)


def _fwd_kernel(*refs, stages, n_in, nb):
    ins, outs = refs[:n_in], refs[n_in:]
    p = ins[0][0]
    ri = 1
    for si, st in enumerate(stages):
        wp = st["wp"]
        for cg in st["convs"]:
            wref = ins[ri]
            b = ins[ri + 1][...]
            mask = ins[ri + 2][...]
            ri += 3
            lo = cg["l_out"]
            base = wp + 1
            acc = None
            for t in range(9):
                dy, dx = divmod(t, 3)
                s = base + (dy - 1) * wp + (dx - 1)
                prod = jnp.dot(wref[t], p[:, s:s + lo],
                               preferred_element_type=jnp.float32)
                acc = prod if acc is None else acc + prod
            acc = jnp.maximum(acc + b, 0.0)
            p = jnp.where(mask > 0.5, acc, 0.0)
        sval = ins[ri][...]
        sden = ins[ri + 1][...]
        bref = ins[ri + 2][...]
        ri += 3
        lm4 = st["lm4"]
        a0 = p[:, 0:lm4]
        a1 = p[:, 1:1 + lm4]
        a2 = p[:, wp:wp + lm4]
        a3 = p[:, wp + 1:wp + 1 + lm4]
        m01 = jnp.maximum(a0, a1)
        k01 = jnp.where(a1 > a0, 1.0, 0.0)
        m23 = jnp.maximum(a2, a3)
        k23 = jnp.where(a3 > a2, 3.0, 2.0)
        m = jnp.maximum(m01, m23)
        k = jnp.where(m23 > m01, k23, k01)
        p = jnp.dot(m, sval, preferred_element_type=jnp.float32)
        kk = jnp.dot(k, sden, preferred_element_type=jnp.float32)
        dy = jnp.where(kk >= 1.5, 1.0, 0.0)
        dx = kk - 2.0 * dy
        idx = bref + dy * float(st["w"]) + dx
        outs[si][0] = (idx + 0.5).astype(jnp.int32)
    w1t, b1c, w2t, b2c, w3t, b3c = (ins[ri + j][...] for j in range(6))
    h = jnp.maximum(jnp.dot(w1t, p, preferred_element_type=jnp.float32)
                    + b1c, 0.0)
    h = jnp.maximum(jnp.dot(w2t, h, preferred_element_type=jnp.float32)
                    + b2c, 0.0)
    o = jnp.dot(w3t, h, preferred_element_type=jnp.float32) + b3c
    outs[5][0] = o.astype(outs[5].dtype)


def kernel(x, feat_0, feat_1, feat_2, feat_3, feat_4, feat_5, feat_6,
           feat_7, feat_8, feat_9, feat_10, feat_11, feat_12, feat_13,
           feat_14, feat_15, feat_16, feat_17, feat_18, feat_19, feat_20,
           feat_21, feat_22, feat_23, feat_24, feat_25, feat_26, feat_27,
           feat_28, feat_29, feat_30, feat_31, feat_32, feat_33, feat_34,
           feat_35, feat_36, feat_37, feat_38, feat_39, feat_40, feat_41,
           feat_42, feat_43, feat_44, feat_45, feat_46, feat_47, feat_48,
           feat_49, feat_50, feat_51, feat_52, feat_53, feat_54, feat_55,
           feat_56, feat_57, w1, b1, w2, b2, w3, b3):
    feats = [feat_0, feat_1, feat_2, feat_3, feat_4, feat_5, feat_6, feat_7,
             feat_8, feat_9, feat_10, feat_11, feat_12, feat_13, feat_14,
             feat_15, feat_16, feat_17, feat_18, feat_19, feat_20, feat_21,
             feat_22, feat_23, feat_24, feat_25, feat_26, feat_27, feat_28,
             feat_29, feat_30, feat_31, feat_32, feat_33, feat_34, feat_35,
             feat_36, feat_37, feat_38, feat_39, feat_40, feat_41, feat_42,
             feat_43, feat_44, feat_45, feat_46, feat_47, feat_48, feat_49,
             feat_50, feat_51, feat_52, feat_53, feat_54, feat_55, feat_56,
             feat_57]
    g0, stages = _geometry()
    n = x.shape[0]
    nb = 8
    eye = jnp.eye(nb, dtype=jnp.float32)

    def bdiag(m):
        r, c = m.shape
        return jnp.einsum("mn,rc->mrnc", eye, m).reshape(nb * r, nb * c)

    ops = []
    fi = 0
    for st in stages:
        for _ in st["convs"]:
            w9, b, mask = feats[fi], feats[fi + 1], feats[fi + 2]
            fi += 3
            coutp, cinp = w9.shape[1], w9.shape[2]
            wbd = jnp.einsum("mn,toc->tmonc", eye, w9).reshape(
                9, nb * coutp, nb * cinp)
            ops += [wbd, jnp.tile(b, (nb, 1)), mask]
        ops += [feats[fi], _dense_gather_np(st["h"], st["w"], st["g_fin"]),
                _dense_base_np(st["h"], st["w"])]
        fi += 2
    ops += [bdiag(w1.T), jnp.tile(b1.reshape(-1, 1), (nb, 1)),
            bdiag(w2.T), jnp.tile(b2.reshape(-1, 1), (nb, 1)),
            bdiag(jnp.pad(w3.T, ((0, 8 - _NCLS), (0, 0)))),
            jnp.tile(jnp.pad(b3, (0, 8 - _NCLS)).reshape(-1, 1), (nb, 1))]
    n_in = 1 + len(ops)

    cp0 = _r8(_IN_CH)
    l0 = 2 * g0 + (_IMG + 2) * (_IMG + 2)
    xp = jnp.pad(x, ((0, 0), (0, cp0 - _IN_CH), (1, 1), (1, 1)))
    xp = jnp.pad(xp.reshape(n, cp0, (_IMG + 2) * (_IMG + 2)),
                 ((0, 0), (0, 0), (g0, g0)))
    xp = xp.reshape(n // nb, nb * cp0, l0)

    out_shapes, out_specs = [], []
    for st in stages:
        cpb = nb * st["cp"]
        hw2 = st["h2"] * st["w2"]
        out_shapes.append(
            jax.ShapeDtypeStruct((n // nb, cpb, hw2), jnp.int32))
        out_specs.append(
            pl.BlockSpec((1, cpb, hw2), lambda i: (i, 0, 0)))
    out_shapes.append(jax.ShapeDtypeStruct((n // nb, nb * 8, 1), jnp.float32))
    out_specs.append(pl.BlockSpec((1, nb * 8, 1), lambda i: (i, 0, 0)))

    in_specs = [pl.BlockSpec((1, nb * cp0, l0), lambda i: (i, 0, 0))]
    for a in ops:
        in_specs.append(
            pl.BlockSpec(a.shape, lambda i, nd=a.ndim: (0,) * nd))

    outs = pl.pallas_call(
        functools.partial(_fwd_kernel, stages=stages, n_in=n_in, nb=nb),
        grid=(n // nb,),
        out_shape=tuple(out_shapes),
        in_specs=in_specs,
        out_specs=tuple(out_specs),
        compiler_params=pltpu.CompilerParams(
            dimension_semantics=("parallel",)),
    )(xp, *ops)

    switch_indices = []
    for st, arr in zip(stages, outs[:5]):
        c, h2, w2 = st["c"], st["h2"], st["w2"]
        switch_indices.append(
            arr.reshape(n, st["cp"], h2 * w2)[:, :c].reshape(n, c, h2, w2))
    logits = outs[5].reshape(n, 8)[:, :_NCLS]
    return logits, switch_indices


# speedup vs baseline: 1.2585x; 1.2345x over previous
import functools

import jax
import jax.numpy as jnp
import numpy as np
from jax.experimental import pallas as pl
from jax.experimental.pallas import tpu as pltpu

_IMG = 32
_IN_CH = 3
_NCLS = 4
_STAGE_CH = [[4, 4], [8, 8], [16, 16, 16, 16], [32, 32, 32, 32],
             [32, 32, 32, 32]]


def _r8(v):
    return ((v + 7) // 8) * 8


def _geometry():
    h = w = _IMG
    g0 = []
    ww = w
    for convs in _STAGE_CH:
        g0.append((len(convs) + 1) * (ww + 3))
        ww //= 2
    stages = []
    c = _IN_CH
    for si, convs in enumerate(_STAGE_CH):
        hp, wp = h + 2, w + 2
        g = g0[si]
        conv_geo = []
        for cout in convs:
            g -= wp + 1
            conv_geo.append(dict(cinp=_r8(c), coutp=_r8(cout),
                                 l_out=2 * g + hp * wp))
            c = cout
        last = si == len(_STAGE_CH) - 1
        h2, w2 = h // 2, w // 2
        stages.append(dict(hp=hp, wp=wp, h=h, w=w, convs=conv_geo,
                           g_fin=g, lm4=2 * g + hp * wp - (wp + 1),
                           cp=_r8(c), c=c, h2=h2, w2=w2, last=last))
        h, w = h2, w2
    return g0[0], stages


def _dense_gather_np(h, w, g):
    hp, wp = h + 2, w + 2
    h2, w2 = h // 2, w // 2
    s = np.zeros((2 * g + hp * wp - (wp + 1), h2 * w2), np.float32)
    for oh in range(h2):
        for ow in range(w2):
            s[g + (2 * oh + 1) * wp + (2 * ow + 1), oh * w2 + ow] = 1.0
    return s


def _dense_base_np(h, w):
    h2, w2 = h // 2, w // 2
    return ((np.arange(h2) * 2 * w)[:, None]
            + np.arange(w2) * 2).astype(np.float32).reshape(1, h2 * w2)


def _fwd_kernel(*refs, stages, n_in, nb):
    ins, outs = refs[:n_in], refs[n_in:]
    p = ins[0][0]
    ri = 1
    for si, st in enumerate(stages):
        wp = st["wp"]
        for cg in st["convs"]:
            wref = ins[ri]
            b = ins[ri + 1][...]
            mask = ins[ri + 2][...]
            ri += 3
            lo = cg["l_out"]
            base = wp + 1
            nbw = cg["nbw"]
            rows = nbw * cg["cinp"]
            accs = []
            for part in range(nb // nbw):
                rs = part * rows
                acc = None
                for t in range(9):
                    dy, dx = divmod(t, 3)
                    s = base + (dy - 1) * wp + (dx - 1)
                    prod = jnp.dot(wref[t], p[rs:rs + rows, s:s + lo],
                                   preferred_element_type=jnp.float32)
                    acc = prod if acc is None else acc + prod
                accs.append(acc)
            acc = accs[0] if len(accs) == 1 else jnp.concatenate(accs, axis=0)
            acc = jnp.maximum(acc + b, 0.0)
            p = jnp.where(mask > 0.5, acc, 0.0)
        sval = ins[ri][...]
        sden = ins[ri + 1][...]
        bref = ins[ri + 2][...]
        ri += 3
        lm4 = st["lm4"]
        a0 = p[:, 0:lm4]
        a1 = p[:, 1:1 + lm4]
        a2 = p[:, wp:wp + lm4]
        a3 = p[:, wp + 1:wp + 1 + lm4]
        m01 = jnp.maximum(a0, a1)
        k01 = jnp.where(a1 > a0, 1.0, 0.0)
        m23 = jnp.maximum(a2, a3)
        k23 = jnp.where(a3 > a2, 3.0, 2.0)
        m = jnp.maximum(m01, m23)
        k = jnp.where(m23 > m01, k23, k01)
        p = jnp.dot(m, sval, preferred_element_type=jnp.float32)
        kk = jnp.dot(k, sden, preferred_element_type=jnp.float32)
        dy = jnp.where(kk >= 1.5, 1.0, 0.0)
        dx = kk - 2.0 * dy
        idx = bref + dy * float(st["w"]) + dx
        outs[si][0] = (idx + 0.5).astype(jnp.int32)
    w1t, b1c, w2t, b2c, w3t, b3c = (ins[ri + j][...] for j in range(6))
    h = jnp.maximum(jnp.dot(w1t, p, preferred_element_type=jnp.float32)
                    + b1c, 0.0)
    h = jnp.maximum(jnp.dot(w2t, h, preferred_element_type=jnp.float32)
                    + b2c, 0.0)
    o = jnp.dot(w3t, h, preferred_element_type=jnp.float32) + b3c
    outs[5][0] = o.astype(outs[5].dtype)


def kernel(x, feat_0, feat_1, feat_2, feat_3, feat_4, feat_5, feat_6,
           feat_7, feat_8, feat_9, feat_10, feat_11, feat_12, feat_13,
           feat_14, feat_15, feat_16, feat_17, feat_18, feat_19, feat_20,
           feat_21, feat_22, feat_23, feat_24, feat_25, feat_26, feat_27,
           feat_28, feat_29, feat_30, feat_31, feat_32, feat_33, feat_34,
           feat_35, feat_36, feat_37, feat_38, feat_39, feat_40, feat_41,
           feat_42, feat_43, feat_44, feat_45, feat_46, feat_47, feat_48,
           feat_49, feat_50, feat_51, feat_52, feat_53, feat_54, feat_55,
           feat_56, feat_57, w1, b1, w2, b2, w3, b3):
    feats = [feat_0, feat_1, feat_2, feat_3, feat_4, feat_5, feat_6, feat_7,
             feat_8, feat_9, feat_10, feat_11, feat_12, feat_13, feat_14,
             feat_15, feat_16, feat_17, feat_18, feat_19, feat_20, feat_21,
             feat_22, feat_23, feat_24, feat_25, feat_26, feat_27, feat_28,
             feat_29, feat_30, feat_31, feat_32, feat_33, feat_34, feat_35,
             feat_36, feat_37, feat_38, feat_39, feat_40, feat_41, feat_42,
             feat_43, feat_44, feat_45, feat_46, feat_47, feat_48, feat_49,
             feat_50, feat_51, feat_52, feat_53, feat_54, feat_55, feat_56,
             feat_57]
    g0, stages = _geometry()
    n = x.shape[0]
    nb = 16

    def bdiag(m, k=nb):
        r, c = m.shape
        ey = jnp.eye(k, dtype=jnp.float32)
        return jnp.einsum("mn,rc->mrnc", ey, m).reshape(k * r, k * c)

    ops = []
    fi = 0
    for st in stages:
        for cg in st["convs"]:
            w9, b, mask = feats[fi], feats[fi + 1], feats[fi + 2]
            fi += 3
            coutp, cinp = w9.shape[1], w9.shape[2]
            nbw = min(nb, 256 // cinp)
            cg["nbw"], cg["cinp"] = nbw, cinp
            ey = jnp.eye(nbw, dtype=jnp.float32)
            wbd = jnp.einsum("mn,toc->tmonc", ey, w9).reshape(
                9, nbw * coutp, nbw * cinp)
            ops += [wbd, jnp.tile(b, (nb, 1)), mask]
        ops += [feats[fi], _dense_gather_np(st["h"], st["w"], st["g_fin"]),
                _dense_base_np(st["h"], st["w"])]
        fi += 2
    ops += [bdiag(w1.T), jnp.tile(b1.reshape(-1, 1), (nb, 1)),
            bdiag(w2.T), jnp.tile(b2.reshape(-1, 1), (nb, 1)),
            bdiag(jnp.pad(w3.T, ((0, 8 - _NCLS), (0, 0)))),
            jnp.tile(jnp.pad(b3, (0, 8 - _NCLS)).reshape(-1, 1), (nb, 1))]
    n_in = 1 + len(ops)

    cp0 = _r8(_IN_CH)
    l0 = 2 * g0 + (_IMG + 2) * (_IMG + 2)
    xp = jnp.pad(x, ((0, 0), (0, cp0 - _IN_CH), (1, 1), (1, 1)))
    xp = jnp.pad(xp.reshape(n, cp0, (_IMG + 2) * (_IMG + 2)),
                 ((0, 0), (0, 0), (g0, g0)))
    xp = xp.reshape(n // nb, nb * cp0, l0)

    out_shapes, out_specs = [], []
    for st in stages:
        cpb = nb * st["cp"]
        hw2 = st["h2"] * st["w2"]
        out_shapes.append(
            jax.ShapeDtypeStruct((n // nb, cpb, hw2), jnp.int32))
        out_specs.append(
            pl.BlockSpec((1, cpb, hw2), lambda i: (i, 0, 0)))
    out_shapes.append(jax.ShapeDtypeStruct((n // nb, nb * 8, 1), jnp.float32))
    out_specs.append(pl.BlockSpec((1, nb * 8, 1), lambda i: (i, 0, 0)))

    in_specs = [pl.BlockSpec((1, nb * cp0, l0), lambda i: (i, 0, 0))]
    for a in ops:
        in_specs.append(
            pl.BlockSpec(a.shape, lambda i, nd=a.ndim: (0,) * nd))

    outs = pl.pallas_call(
        functools.partial(_fwd_kernel, stages=stages, n_in=n_in, nb=nb),
        grid=(n // nb,),
        out_shape=tuple(out_shapes),
        in_specs=in_specs,
        out_specs=tuple(out_specs),
        compiler_params=pltpu.CompilerParams(
            dimension_semantics=("parallel",)),
    )(xp, *ops)

    switch_indices = []
    for st, arr in zip(stages, outs[:5]):
        c, h2, w2 = st["c"], st["h2"], st["w2"]
        switch_indices.append(
            arr.reshape(n, st["cp"], h2 * w2)[:, :c].reshape(n, c, h2, w2))
    logits = outs[5].reshape(n, 8)[:, :_NCLS]
    return logits, switch_indices
